# baseline (speedup 1.0000x reference)
import jax
import jax.numpy as jnp
from jax import lax
from jax.experimental import pallas as pl
from jax.experimental.pallas import tpu as pltpu

_B = 16384
_C = 1000
_BR = 2048
_BC = 512
_NB = _B // _BR
_NC = 2
_K = _B // 2
_NEG = -3.0e38


def _fused_body(x_ref, t_ref, out_ref, m_scr, s_scr, g_scr, loss_scr):
    i = pl.program_id(0)
    j = pl.program_id(1)
    x = x_ref[...]
    t = t_ref[...]
    col = j * _BC + lax.broadcasted_iota(jnp.int32, x.shape, 1)
    valid = col < _C
    xm = jnp.where(valid, x, _NEG)
    m = jnp.max(xm, axis=1)
    s = jnp.sum(jnp.where(valid, jnp.exp(x - m[:, None]), 0.0), axis=1)
    tgt = jnp.sum(jnp.where(col == t[:, None], xm, 0.0), axis=1)

    @pl.when(j == 0)
    def _first():
        m_scr[...] = m
        s_scr[...] = s
        g_scr[...] = tgt

    @pl.when(j == _NC - 1)
    def _merge():
        m0 = m_scr[...]
        s0 = s_scr[...]
        mm = jnp.maximum(m0, m)
        ss = s0 * jnp.exp(m0 - mm) + s * jnp.exp(m - mm)
        loss_scr[pl.ds(i * _BR, _BR)] = mm + jnp.log(ss) - (g_scr[...] + tgt)

    @pl.when((i == _NB - 1) & (j == _NC - 1))
    def _select():
        v_all = loss_scr[...]
        bits = lax.bitcast_convert_type(v_all, jnp.int32)
        ukey = lax.bitcast_convert_type(
            jnp.where(bits < 0, ~bits, bits | jnp.int32(-2147483648)), jnp.uint32
        )

        def step(b, p):
            c = p | (jnp.uint32(1) << (jnp.uint32(31) - b.astype(jnp.uint32)))
            cnt = jnp.sum((ukey >= c).astype(jnp.int32))
            return jnp.where(cnt >= _K, c, p)

        p = lax.fori_loop(0, 32, step, jnp.uint32(0))
        pi = lax.bitcast_convert_type(p, jnp.int32)
        vbits = jnp.where(pi < 0, pi & jnp.int32(0x7FFFFFFF), ~pi)
        v = lax.bitcast_convert_type(vbits, jnp.float32)
        sel = ukey > p
        cnt_gt = jnp.sum(sel.astype(jnp.int32))
        ssum = jnp.sum(jnp.where(sel, v_all, 0.0))
        rem = (_K - cnt_gt).astype(jnp.float32)
        out_ref[0, 0] = ssum + jnp.where(cnt_gt == _K, 0.0, rem * v)


@jax.jit
def kernel(input, target):
    out = pl.pallas_call(
        _fused_body,
        grid=(_NB, _NC),
        in_specs=[
            pl.BlockSpec((_BR, _BC), lambda i, j: (i, j)),
            pl.BlockSpec((_BR,), lambda i, j: (i,)),
        ],
        out_specs=pl.BlockSpec(memory_space=pltpu.SMEM),
        out_shape=jax.ShapeDtypeStruct((1, 1), jnp.float32),
        scratch_shapes=[
            pltpu.VMEM((_BR,), jnp.float32),
            pltpu.VMEM((_BR,), jnp.float32),
            pltpu.VMEM((_BR,), jnp.float32),
            pltpu.VMEM((_B,), jnp.float32),
        ],
    )(input, target)
    return out[0, 0]


# speedup vs baseline: 1.4180x; 1.4180x over previous
import jax
import jax.numpy as jnp
from jax import lax
from jax.experimental import pallas as pl
from jax.experimental.pallas import tpu as pltpu

_B = 16384
_C = 1000
_BR = 2048
_NB = _B // _BR
_K = _B // 2


def _fused_body(x_ref, t_ref, out_ref, loss_scr):
    i = pl.program_id(0)
    x = x_ref[...]
    t = t_ref[...]
    m = jnp.max(x, axis=1)
    s = jnp.sum(jnp.exp(x - m[:, None]), axis=1)
    lse = m + jnp.log(s)
    col = lax.broadcasted_iota(jnp.int32, x.shape, 1)
    tgt = jnp.sum(jnp.where(col == t[:, None], x, 0.0), axis=1)
    loss_scr[pl.ds(i * _BR, _BR)] = lse - tgt

    @pl.when(i == _NB - 1)
    def _select():
        v_all = loss_scr[...]
        bits = lax.bitcast_convert_type(v_all, jnp.int32)
        ukey = lax.bitcast_convert_type(
            jnp.where(bits < 0, ~bits, bits | jnp.int32(-2147483648)), jnp.uint32
        )

        def step(j, p):
            c = p | (jnp.uint32(1) << (jnp.uint32(31) - j.astype(jnp.uint32)))
            cnt = jnp.sum((ukey >= c).astype(jnp.int32))
            return jnp.where(cnt >= _K, c, p)

        p = lax.fori_loop(0, 32, step, jnp.uint32(0))
        pi = lax.bitcast_convert_type(p, jnp.int32)
        vbits = jnp.where(pi < 0, pi & jnp.int32(0x7FFFFFFF), ~pi)
        v = lax.bitcast_convert_type(vbits, jnp.float32)
        sel = ukey > p
        cnt_gt = jnp.sum(sel.astype(jnp.int32))
        ssum = jnp.sum(jnp.where(sel, v_all, 0.0))
        rem = (_K - cnt_gt).astype(jnp.float32)
        out_ref[0, 0] = ssum + jnp.where(cnt_gt == _K, 0.0, rem * v)


@jax.jit
def kernel(input, target):
    out = pl.pallas_call(
        _fused_body,
        grid=(_NB,),
        in_specs=[
            pl.BlockSpec((_BR, _C), lambda i: (i, 0)),
            pl.BlockSpec((_BR,), lambda i: (i,)),
        ],
        out_specs=pl.BlockSpec(memory_space=pltpu.SMEM),
        out_shape=jax.ShapeDtypeStruct((1, 1), jnp.float32),
        scratch_shapes=[pltpu.VMEM((_B,), jnp.float32)],
    )(input, target)
    return out[0, 0]


# speedup vs baseline: 1.4346x; 1.0117x over previous
import jax
import jax.numpy as jnp
from jax import lax
from jax.experimental import pallas as pl
from jax.experimental.pallas import tpu as pltpu

_B = 16384
_C = 1000
_BR = 2048
_K = _B // 2


def _loss_body(x_ref, t_ref, loss_ref):
    x = x_ref[...]
    t = t_ref[...]
    m = jnp.max(x, axis=1)
    s = jnp.sum(jnp.exp(x - m[:, None]), axis=1)
    lse = m + jnp.log(s)
    col = lax.broadcasted_iota(jnp.int32, x.shape, 1)
    tgt = jnp.sum(jnp.where(col == t[:, None], x, 0.0), axis=1)
    loss_ref[...] = lse - tgt


def _topk_sum_body(loss_ref, out_ref):
    x = loss_ref[...]
    bits = lax.bitcast_convert_type(x, jnp.int32)
    ukey = lax.bitcast_convert_type(
        jnp.where(bits < 0, ~bits, bits | jnp.int32(-2147483648)), jnp.uint32
    )

    def step(i, p):
        c = p | (jnp.uint32(1) << (jnp.uint32(31) - i.astype(jnp.uint32)))
        cnt = jnp.sum((ukey >= c).astype(jnp.int32))
        return jnp.where(cnt >= _K, c, p)

    p = lax.fori_loop(0, 32, step, jnp.uint32(0))
    pi = lax.bitcast_convert_type(p, jnp.int32)
    vbits = jnp.where(pi < 0, pi & jnp.int32(0x7FFFFFFF), ~pi)
    v = lax.bitcast_convert_type(vbits, jnp.float32)
    sel = ukey > p
    cnt_gt = jnp.sum(sel.astype(jnp.int32))
    s = jnp.sum(jnp.where(sel, x, 0.0))
    rem = (_K - cnt_gt).astype(jnp.float32)
    out_ref[0, 0] = s + jnp.where(cnt_gt == _K, 0.0, rem * v)


@jax.jit
def kernel(input, target):
    loss = pl.pallas_call(
        _loss_body,
        grid=(_B // _BR,),
        in_specs=[
            pl.BlockSpec((_BR, _C), lambda i: (i, 0)),
            pl.BlockSpec((_BR,), lambda i: (i,)),
        ],
        out_specs=pl.BlockSpec((_BR,), lambda i: (i,)),
        out_shape=jax.ShapeDtypeStruct((_B,), jnp.float32),
    )(input, target)

    out = pl.pallas_call(
        _topk_sum_body,
        out_shape=jax.ShapeDtypeStruct((1, 1), jnp.float32),
        out_specs=pl.BlockSpec(memory_space=pltpu.SMEM),
    )(loss.reshape(128, 128))
    return out[0, 0]
